# Initial kernel scaffold; baseline (speedup 1.0000x reference)
import jax
import jax.numpy as jnp
from jax.experimental import pallas as pl

_N = 8192
_NB = 4
_K = 16
_GAMMA = 2.0
_LOSS_WEIGHT = 1.0
_BLK = _N // _NB
_QT = 256
_QPB = _BLK // _QT


def _prep_body(xyzT_ref, tm_ref, outT_ref, cT_ref, oT_ref):
    x = xyzT_ref[0:1, :]
    y = xyzT_ref[1:2, :]
    z = xyzT_ref[2:3, :]
    for r in range(3):
        t0 = tm_ref[4 * r + 0:4 * r + 1, :]
        t1 = tm_ref[4 * r + 1:4 * r + 2, :]
        t2 = tm_ref[4 * r + 2:4 * r + 3, :]
        t3 = tm_ref[4 * r + 3:4 * r + 4, :]
        cT_ref[r:r + 1, :] = t0 * x + t1 * y + t2 * z + t3
    ox = outT_ref[0:1, :]
    oy = outT_ref[1:2, :]
    oz = outT_ref[2:3, :]
    denom = jnp.maximum(jnp.sqrt(ox * ox + oy * oy + oz * oz), 1e-12)
    ox = ox / denom
    oy = oy / denom
    oz = oz / denom
    for r in range(3):
        t0 = tm_ref[4 * r + 0:4 * r + 1, :]
        t1 = tm_ref[4 * r + 1:4 * r + 2, :]
        t2 = tm_ref[4 * r + 2:4 * r + 3, :]
        oT_ref[r:r + 1, :] = t0 * ox + t1 * oy + t2 * oz


def _main_body(qc_ref, qo_ref, ccT_ref, coT_ref, acc_ref):
    i = pl.program_id(0)

    @pl.when(i == 0)
    def _init():
        acc_ref[0, 0] = 0.0

    qx = qc_ref[:, 0:1]
    qy = qc_ref[:, 1:2]
    qz = qc_ref[:, 2:3]
    cx = ccT_ref[0:1, :]
    cy = ccT_ref[1:2, :]
    cz = ccT_ref[2:3, :]
    dx = qx - cx
    dy = qy - cy
    dz = qz - cz
    d2 = dx * dx + dy * dy + dz * dz

    l1 = (jnp.abs(qo_ref[:, 0:1] - coT_ref[0:1, :])
          + jnp.abs(qo_ref[:, 1:2] - coT_ref[1:2, :])
          + jnp.abs(qo_ref[:, 2:3] - coT_ref[2:3, :]))

    t = jnp.full((_QT, 1), -jnp.inf, dtype=jnp.float32)
    for _ in range(_K):
        t = jnp.min(jnp.where(d2 > t, d2, jnp.inf), axis=1, keepdims=True)

    val = jnp.exp(d2 * (-1.0 / (_GAMMA * _GAMMA))) * l1
    lt = d2 < t
    eq = d2 == t
    n_lt = jnp.sum(lt.astype(jnp.float32), axis=1, keepdims=True)
    n_eq = jnp.sum(eq.astype(jnp.float32), axis=1, keepdims=True)
    s_lt = jnp.sum(jnp.where(lt, val, 0.0), axis=1, keepdims=True)
    s_eq = jnp.sum(jnp.where(eq, val, 0.0), axis=1, keepdims=True)
    factor = jnp.clip(_K - n_lt, 0.0, n_eq) / jnp.maximum(n_eq, 1.0)
    rows = s_lt + s_eq * factor
    acc_ref[0, 0] += jnp.sum(rows) * (_LOSS_WEIGHT / (_N * _K))


def kernel(coord, intensity, out, target, untransform_coord):
    del intensity, target
    xyzT = coord[:, 1:4].T.astype(jnp.float32)
    tm = untransform_coord.reshape(_N, 16).T.astype(jnp.float32)
    outT = out.T.astype(jnp.float32)

    cT, oT = pl.pallas_call(
        _prep_body,
        out_shape=[jax.ShapeDtypeStruct((3, _N), jnp.float32),
                   jax.ShapeDtypeStruct((3, _N), jnp.float32)],
    )(xyzT, tm, outT)

    c = cT.T
    o = oT.T

    nprog = _N // _QT
    acc = pl.pallas_call(
        _main_body,
        grid=(nprog,),
        in_specs=[
            pl.BlockSpec((_QT, 3), lambda i: (i, 0)),
            pl.BlockSpec((_QT, 3), lambda i: (i, 0)),
            pl.BlockSpec((3, _BLK), lambda i: (0, (i // _QPB) ^ 1)),
            pl.BlockSpec((3, _BLK), lambda i: (0, (i // _QPB) ^ 1)),
        ],
        out_specs=pl.BlockSpec((1, 1), lambda i: (0, 0)),
        out_shape=jax.ShapeDtypeStruct((1, 1), jnp.float32),
    )(c, o, cT, oT)
    return acc[0, 0]


# speedup vs baseline: 49.1691x; 49.1691x over previous
import jax
import jax.numpy as jnp
from jax.experimental import pallas as pl

_N = 8192
_NB = 4
_K = 16
_GAMMA = 2.0
_LOSS_WEIGHT = 1.0
_BLK = _N // _NB
_QT = 256
_QPB = _BLK // _QT


def _prep_body(xyzT_ref, tm_ref, outT_ref, cT_ref, oT_ref):
    x = xyzT_ref[0:1, :]
    y = xyzT_ref[1:2, :]
    z = xyzT_ref[2:3, :]
    for r in range(3):
        t0 = tm_ref[4 * r + 0:4 * r + 1, :]
        t1 = tm_ref[4 * r + 1:4 * r + 2, :]
        t2 = tm_ref[4 * r + 2:4 * r + 3, :]
        t3 = tm_ref[4 * r + 3:4 * r + 4, :]
        cT_ref[r:r + 1, :] = t0 * x + t1 * y + t2 * z + t3
    ox = outT_ref[0:1, :]
    oy = outT_ref[1:2, :]
    oz = outT_ref[2:3, :]
    denom = jnp.maximum(jnp.sqrt(ox * ox + oy * oy + oz * oz), 1e-12)
    ox = ox / denom
    oy = oy / denom
    oz = oz / denom
    for r in range(3):
        t0 = tm_ref[4 * r + 0:4 * r + 1, :]
        t1 = tm_ref[4 * r + 1:4 * r + 2, :]
        t2 = tm_ref[4 * r + 2:4 * r + 3, :]
        oT_ref[r:r + 1, :] = t0 * ox + t1 * oy + t2 * oz


def _main_body(qc_ref, qo_ref, ccT_ref, coT_ref, acc_ref):
    i = pl.program_id(0)

    @pl.when(i == 0)
    def _init():
        acc_ref[:, :] = jnp.zeros((1, 1), dtype=jnp.float32)

    qx = qc_ref[:, 0:1]
    qy = qc_ref[:, 1:2]
    qz = qc_ref[:, 2:3]
    cx = ccT_ref[0:1, :]
    cy = ccT_ref[1:2, :]
    cz = ccT_ref[2:3, :]
    dx = qx - cx
    dy = qy - cy
    dz = qz - cz
    d2 = dx * dx + dy * dy + dz * dz

    l1 = (jnp.abs(qo_ref[:, 0:1] - coT_ref[0:1, :])
          + jnp.abs(qo_ref[:, 1:2] - coT_ref[1:2, :])
          + jnp.abs(qo_ref[:, 2:3] - coT_ref[2:3, :]))

    t = jnp.full((_QT, 1), -jnp.inf, dtype=jnp.float32)
    for _ in range(_K):
        t = jnp.min(jnp.where(d2 > t, d2, jnp.inf), axis=1, keepdims=True)

    val = jnp.exp(d2 * (-1.0 / (_GAMMA * _GAMMA))) * l1
    lt = d2 < t
    eq = d2 == t
    n_lt = jnp.sum(lt.astype(jnp.float32), axis=1, keepdims=True)
    n_eq = jnp.sum(eq.astype(jnp.float32), axis=1, keepdims=True)
    s_lt = jnp.sum(jnp.where(lt, val, 0.0), axis=1, keepdims=True)
    s_eq = jnp.sum(jnp.where(eq, val, 0.0), axis=1, keepdims=True)
    factor = jnp.clip(_K - n_lt, 0.0, n_eq) / jnp.maximum(n_eq, 1.0)
    rows = s_lt + s_eq * factor
    part = jnp.sum(rows, keepdims=True).reshape(1, 1) * (_LOSS_WEIGHT / (_N * _K))
    acc_ref[:, :] = acc_ref[:, :] + part


def kernel(coord, intensity, out, target, untransform_coord):
    del intensity, target
    xyzT = coord[:, 1:4].T.astype(jnp.float32)
    tm = untransform_coord.reshape(_N, 16).T.astype(jnp.float32)
    outT = out.T.astype(jnp.float32)

    cT, oT = pl.pallas_call(
        _prep_body,
        out_shape=[jax.ShapeDtypeStruct((3, _N), jnp.float32),
                   jax.ShapeDtypeStruct((3, _N), jnp.float32)],
    )(xyzT, tm, outT)

    c = cT.T
    o = oT.T

    nprog = _N // _QT
    acc = pl.pallas_call(
        _main_body,
        grid=(nprog,),
        in_specs=[
            pl.BlockSpec((_QT, 3), lambda i: (i, 0)),
            pl.BlockSpec((_QT, 3), lambda i: (i, 0)),
            pl.BlockSpec((3, _BLK), lambda i: (0, (i // _QPB) ^ 1)),
            pl.BlockSpec((3, _BLK), lambda i: (0, (i // _QPB) ^ 1)),
        ],
        out_specs=pl.BlockSpec((1, 1), lambda i: (0, 0)),
        out_shape=jax.ShapeDtypeStruct((1, 1), jnp.float32),
    )(c, o, cT, oT)
    return acc[0, 0]
